# Initial kernel scaffold; baseline (speedup 1.0000x reference)
import jax
import jax.numpy as jnp
from jax.experimental import pallas as pl


def kernel(adj, x, pseudo, Ws, roots, biases, lin1_w, lin1_b, lin2_w, lin2_b):
    raise NotImplementedError("write your pallas kernel here")


# speedup vs baseline: 1.0306x; 1.0306x over previous
import functools

import jax
import jax.numpy as jnp
import numpy as np
from jax.experimental import pallas as pl
from jax.experimental.pallas import tpu as pltpu

KS = 5
DIM = 3
K3 = KS ** DIM
_BITS = np.array([[(s >> d) & 1 for d in range(DIM)] for s in range(2 ** DIM)],
                 dtype=np.int32)
_STRIDES = np.array([1, KS, KS * KS], dtype=np.int32)


def _spline_basis(pseudo):
    v = pseudo * (KS - 1)
    bot = jnp.floor(v)
    frac = v - bot
    bits = jnp.asarray(_BITS)
    f = frac[:, None, :]
    b = jnp.where(bits[None, :, :] == 1, f, 1.0 - f)
    basis = jnp.prod(b, axis=-1)
    idx = jnp.clip(bot.astype(jnp.int32)[:, None, :] + bits[None, :, :], 0, KS - 1)
    kidx = jnp.sum(idx * jnp.asarray(_STRIDES), axis=-1)
    return basis, kidx


def _combine_kernel(t_ref, w_ref, x_ref, root_ref, bias_ref, invdeg_ref, o_ref):
    k = pl.program_id(0)
    nk = pl.num_programs(0)

    @pl.when(k == 0)
    def _init():
        o_ref[...] = jnp.zeros_like(o_ref)

    o_ref[...] += jnp.dot(t_ref[0], w_ref[0],
                          preferred_element_type=jnp.float32)

    @pl.when(k == nk - 1)
    def _fin():
        acc = o_ref[...] * invdeg_ref[...]
        acc = acc + jnp.dot(x_ref[...], root_ref[...],
                            preferred_element_type=jnp.float32)
        acc = acc + bias_ref[...]
        o_ref[...] = jnp.where(acc > 0, acc, jnp.exp(acc) - 1.0)


def _combine(T, W, x, root, bias, invdeg):
    N, din = x.shape
    dout = W.shape[-1]
    return pl.pallas_call(
        _combine_kernel,
        grid=(K3,),
        in_specs=[
            pl.BlockSpec((1, N, din), lambda k: (k, 0, 0)),
            pl.BlockSpec((1, din, dout), lambda k: (k, 0, 0)),
            pl.BlockSpec((N, din), lambda k: (0, 0)),
            pl.BlockSpec((din, dout), lambda k: (0, 0)),
            pl.BlockSpec((1, dout), lambda k: (0, 0)),
            pl.BlockSpec((N, 1), lambda k: (0, 0)),
        ],
        out_specs=pl.BlockSpec((N, dout), lambda k: (0, 0)),
        out_shape=jax.ShapeDtypeStruct((N, dout), jnp.float32),
    )(T, W, x, root, bias.reshape(1, -1), invdeg)


def _mlp_kernel(h_ref, w1_ref, b1_ref, w2_ref, b2_ref, o_ref):
    h = jnp.dot(h_ref[...], w1_ref[...], preferred_element_type=jnp.float32)
    h = h + b1_ref[...]
    h = jnp.where(h > 0, h, jnp.exp(h) - 1.0)
    z = jnp.dot(h, w2_ref[...], preferred_element_type=jnp.float32)
    z = z + b2_ref[...]
    m = jnp.max(z, axis=-1, keepdims=True)
    zs = z - m
    lse = jnp.log(jnp.sum(jnp.exp(zs), axis=-1, keepdims=True))
    o_ref[...] = zs - lse


def _mlp(h, w1, b1, w2, b2):
    N = h.shape[0]
    return pl.pallas_call(
        _mlp_kernel,
        out_shape=jax.ShapeDtypeStruct((N, w2.shape[1]), jnp.float32),
    )(h, w1, b1.reshape(1, -1), w2, b2.reshape(1, -1))


def kernel(adj, x, pseudo, Ws, roots, biases, lin1_w, lin1_b, lin2_w, lin2_b):
    src = adj[0]
    dst = adj[1]
    E = src.shape[0]
    N = x.shape[0]
    basis, kidx = _spline_basis(pseudo)
    flat = (kidx * N + dst[:, None]).reshape(-1)
    deg = jax.ops.segment_sum(jnp.ones((E,), jnp.float32), dst, num_segments=N)
    invdeg = (1.0 / jnp.maximum(deg, 1.0))[:, None]

    h = x
    for W, root, b in zip(Ws, roots, biases):
        din = h.shape[1]
        x_src = jnp.take(h, src, axis=0)
        vals = (basis[:, :, None] * x_src[:, None, :]).reshape(E * 8, din)
        T = jax.ops.segment_sum(vals, flat, num_segments=K3 * N).reshape(K3, N, din)
        h = _combine(T, W, h, root, b, invdeg)
    return _mlp(h, lin1_w, lin1_b, lin2_w, lin2_b)


# speedup vs baseline: 3.0953x; 3.0033x over previous
import functools

import jax
import jax.numpy as jnp
import numpy as np
from jax import lax
from jax.experimental import pallas as pl
from jax.experimental.pallas import tpu as pltpu
from jax.experimental.pallas import tpu_sc as plsc

KS = 5
DIM = 3
K3 = KS ** DIM

NC = 2
NS = 16
LANES = 16
CHUNK = 128
WIDTH = 128


def _basis_kernel(p_ref, basis_ref, kidx_ref):
    v = p_ref[...] * (KS - 1)
    bot = jnp.floor(v)
    frac = v - bot
    boti = bot.astype(jnp.int32)
    bcols = []
    kcols = []
    for s in range(8):
        w = None
        ki = None
        for d in range(DIM):
            bit = (s >> d) & 1
            f = frac[:, d:d + 1]
            wd = f if bit else 1.0 - f
            w = wd if w is None else w * wd
            idx = jnp.clip(boti[:, d:d + 1] + bit, 0, KS - 1)
            contrib = idx * (KS ** d)
            ki = contrib if ki is None else ki + contrib
        bcols.append(w)
        kcols.append(ki)
    basis_ref[...] = jnp.concatenate(bcols, axis=1)
    kidx_ref[...] = jnp.concatenate(kcols, axis=1)


def _spline_basis(pseudo):
    E = pseudo.shape[0]
    B = 2000
    return pl.pallas_call(
        _basis_kernel,
        grid=(E // B,),
        in_specs=[pl.BlockSpec((B, DIM), lambda i: (i, 0))],
        out_specs=[pl.BlockSpec((B, 8), lambda i: (i, 0)),
                   pl.BlockSpec((B, 8), lambda i: (i, 0))],
        out_shape=[jax.ShapeDtypeStruct((E, 8), jnp.float32),
                   jax.ShapeDtypeStruct((E, 8), jnp.int32)],
    )(pseudo)


def _elu(x):
    return jnp.where(x > 0, x, jnp.exp(x) - 1.0)


def _expand0_kernel(x0_ref, x1_ref, x2_ref, x3_ref, w_ref, y_ref):
    w = w_ref[0]
    y_ref[...] = jnp.concatenate(
        [x0_ref[...] * w, x1_ref[...] * w, x2_ref[...] * w, x3_ref[...] * w],
        axis=1)


def _expand0(xs, W):
    NTP = xs[0].shape[0]
    return pl.pallas_call(
        _expand0_kernel,
        grid=(K3,),
        in_specs=[pl.BlockSpec((NTP, 1), lambda k: (0, 0))] * 4
        + [pl.BlockSpec((1, 1, 32), lambda k: (k, 0, 0))],
        out_specs=pl.BlockSpec((NTP, WIDTH), lambda k: (k, 0)),
        out_shape=jax.ShapeDtypeStruct((K3 * NTP, WIDTH), jnp.float32),
    )(*xs, W)


def _expand2_kernel(he_ref, ho_ref, w_ref, y_ref):
    w = w_ref[0]
    ye = jnp.dot(he_ref[...], w, preferred_element_type=jnp.float32)
    yo = jnp.dot(ho_ref[...], w, preferred_element_type=jnp.float32)
    y_ref[...] = jnp.concatenate([ye, yo], axis=1)


def _expand2(he, ho, W):
    NH, din = he.shape
    return pl.pallas_call(
        _expand2_kernel,
        grid=(K3,),
        in_specs=[pl.BlockSpec((NH, din), lambda k: (0, 0)),
                  pl.BlockSpec((NH, din), lambda k: (0, 0)),
                  pl.BlockSpec((1, din, 64), lambda k: (k, 0, 0))],
        out_specs=pl.BlockSpec((NH, WIDTH), lambda k: (k, 0)),
        out_shape=jax.ShapeDtypeStruct((K3 * NH, WIDTH), jnp.float32),
    )(he, ho, W)


def _fold(msg_blk, pack, dout):
    msum = msg_blk[0] + msg_blk[1]
    acc = msum[:, 0:dout]
    for j in range(1, pack):
        acc = acc + msum[:, j * dout:(j + 1) * dout]
    return acc


def _combine_kernel(pack, dout, msg_ref, deg_ref, hprev_ref, root_ref,
                    bias_ref, h_ref, he_ref, ho_ref):
    m = _fold(msg_ref[...], pack, dout)
    dsum = deg_ref[0] + deg_ref[1]
    invd = 1.0 / jnp.maximum(dsum, 1.0)
    h = m * invd + jnp.dot(hprev_ref[...], root_ref[...],
                           preferred_element_type=jnp.float32)
    h = _elu(h + bias_ref[...])
    h_ref[...] = h
    nb = h.shape[0]
    h3 = h.reshape(nb // 2, 2, dout)
    he_ref[...] = h3[:, 0, :]
    ho_ref[...] = h3[:, 1, :]


def _combine(msg, deg, hprev, root, bias, pack, dout):
    N = hprev.shape[0]
    dpp = hprev.shape[1]
    NB = 400
    kfn = functools.partial(_combine_kernel, pack, dout)
    return pl.pallas_call(
        kfn,
        grid=(N // NB,),
        in_specs=[
            pl.BlockSpec((2, NB, WIDTH), lambda i: (0, i, 0)),
            pl.BlockSpec((2, NB, 1), lambda i: (0, i, 0)),
            pl.BlockSpec((NB, dpp), lambda i: (i, 0)),
            pl.BlockSpec((dpp, dout), lambda i: (0, 0)),
            pl.BlockSpec((1, dout), lambda i: (0, 0)),
        ],
        out_specs=[pl.BlockSpec((NB, dout), lambda i: (i, 0)),
                   pl.BlockSpec((NB // 2, dout), lambda i: (i, 0)),
                   pl.BlockSpec((NB // 2, dout), lambda i: (i, 0))],
        out_shape=[jax.ShapeDtypeStruct((N, dout), jnp.float32),
                   jax.ShapeDtypeStruct((N // 2, dout), jnp.float32),
                   jax.ShapeDtypeStruct((N // 2, dout), jnp.float32)],
    )(msg, deg, hprev, root, bias.reshape(1, -1))


def _final_kernel(pack, dout, msg_ref, deg_ref, hprev_ref, root_ref, bias_ref,
                  w1_ref, b1_ref, w2_ref, b2_ref, o_ref):
    m = _fold(msg_ref[...], pack, dout)
    dsum = deg_ref[0] + deg_ref[1]
    invd = 1.0 / jnp.maximum(dsum, 1.0)
    h = m * invd + jnp.dot(hprev_ref[...], root_ref[...],
                           preferred_element_type=jnp.float32)
    h = _elu(h + bias_ref[...])
    h = _elu(jnp.dot(h, w1_ref[...], preferred_element_type=jnp.float32)
             + b1_ref[...])
    z = jnp.dot(h, w2_ref[...], preferred_element_type=jnp.float32) + b2_ref[...]
    mx = jnp.max(z, axis=-1, keepdims=True)
    zs = z - mx
    lse = jnp.log(jnp.sum(jnp.exp(zs), axis=-1, keepdims=True))
    o_ref[...] = zs - lse


def _final(msg, deg, hprev, root, bias, w1, b1, w2, b2, pack, dout):
    N = hprev.shape[0]
    dpp = hprev.shape[1]
    NB = 400
    kfn = functools.partial(_final_kernel, pack, dout)
    return pl.pallas_call(
        kfn,
        grid=(N // NB,),
        in_specs=[
            pl.BlockSpec((2, NB, WIDTH), lambda i: (0, i, 0)),
            pl.BlockSpec((2, NB, 1), lambda i: (0, i, 0)),
            pl.BlockSpec((NB, dpp), lambda i: (i, 0)),
            pl.BlockSpec((dpp, dout), lambda i: (0, 0)),
            pl.BlockSpec((1, dout), lambda i: (0, 0)),
            pl.BlockSpec(w1.shape, lambda i: (0, 0)),
            pl.BlockSpec((1, w1.shape[1]), lambda i: (0, 0)),
            pl.BlockSpec(w2.shape, lambda i: (0, 0)),
            pl.BlockSpec((1, w2.shape[1]), lambda i: (0, 0)),
        ],
        out_specs=pl.BlockSpec((NB, w2.shape[1]), lambda i: (i, 0)),
        out_shape=jax.ShapeDtypeStruct((N, w2.shape[1]), jnp.float32),
    )(msg, deg, hprev, root, bias.reshape(1, -1), w1, b1.reshape(1, -1),
      w2, b2.reshape(1, -1))


def _bcast_lane(vec, i):
    return lax.broadcast_in_dim(lax.slice(vec, (i,), (i + 1,)), (LANES,), (0,))


def _npad(n_nodes):
    g = NS * 8
    return ((n_nodes + g - 1) // g) * g


def _sc_msg_body(n_nodes, pack, cpw, with_deg, *refs):
    if with_deg:
        (y_hbm, meta_hbm, bas_hbm, zeros_hbm, zeros1_hbm, out_hbm,
         deg0_hbm, deg1_hbm,
         metav, basv, rows, basisf, acc, dacc, sem) = refs
    else:
        (y_hbm, meta_hbm, bas_hbm, zeros_hbm, out_hbm,
         metav, basv, rows, acc, sem) = refs
    c = lax.axis_index("c")
    s = lax.axis_index("s")
    wid = s * NC + c
    rpt = _npad(n_nodes) // NS
    dout = WIDTH // pack

    pltpu.sync_copy(zeros_hbm.at[pl.ds(s * rpt, rpt)],
                    acc.at[pl.ds(s * rpt, rpt)])
    if with_deg:
        @pl.when(s == 0)
        def _zero_deg():
            pltpu.sync_copy(zeros1_hbm, dacc)
    plsc.subcore_barrier()

    def chunk_body(i, carry):
        base = wid * cpw + i
        pltpu.sync_copy(meta_hbm.at[base], metav)
        pltpu.sync_copy(bas_hbm.at[base], basv)
        pltpu.async_copy(y_hbm.at[metav.at[0]], rows, sem).wait()
        for g in range(CHUNK // LANES):
            bvs = [basv[j, pl.ds(g * LANES, LANES)] for j in range(pack)]
            if with_deg:
                bsum = bvs[0]
                for j in range(1, pack):
                    bsum = bsum + bvs[j]
                basisf[pl.ds(g * LANES, LANES)] = bsum
            for i2 in range(LANES):
                r = g * LANES + i2
                for j in range(pack):
                    b = _bcast_lane(bvs[j], i2)
                    for t in range(dout // LANES):
                        col = j * dout + t * LANES
                        rows[r, pl.ds(col, LANES)] = (
                            rows[r, pl.ds(col, LANES)] * b)
        pltpu.sync_copy(rows, acc.at[metav.at[1]], add=True)
        if with_deg:
            pltpu.sync_copy(basisf, dacc.at[metav.at[1]], add=True)
        return carry

    lax.fori_loop(0, cpw, chunk_body, 0)
    plsc.subcore_barrier()

    pltpu.sync_copy(acc.at[pl.ds(s * rpt, rpt)],
                    out_hbm.at[c, pl.ds(s * rpt, rpt)])
    if with_deg:
        @pl.when((s == 0) & (c == 0))
        def _copy_deg0():
            pltpu.sync_copy(dacc, deg0_hbm)

        @pl.when((s == 0) & (c == 1))
        def _copy_deg1():
            pltpu.sync_copy(dacc, deg1_hbm)


@functools.partial(jax.jit, static_argnums=(4, 5))
def _sc_msg(yflat, meta, bas, zeros2d, n_nodes, pack):
    nchunks = meta.shape[0]
    cpw = nchunks // (NC * NS)
    npad = _npad(n_nodes)
    mesh = plsc.VectorSubcoreMesh(core_axis_name="c", subcore_axis_name="s")
    body = functools.partial(_sc_msg_body, n_nodes, pack, cpw, False)
    f = pl.kernel(
        body,
        out_type=jax.ShapeDtypeStruct((NC, npad, WIDTH), jnp.float32),
        mesh=mesh,
        scratch_types=[
            pltpu.VMEM((2, CHUNK), jnp.int32),
            pltpu.VMEM((pack, CHUNK), jnp.float32),
            pltpu.VMEM((CHUNK, WIDTH), jnp.float32),
            pltpu.VMEM_SHARED((npad, WIDTH), jnp.float32),
            pltpu.SemaphoreType.DMA,
        ],
    )
    return f(yflat, meta, bas, zeros2d)


@functools.partial(jax.jit, static_argnums=(5, 6))
def _sc_msg_deg(yflat, meta, bas, zeros2d, zeros1, n_nodes, pack):
    nchunks = meta.shape[0]
    cpw = nchunks // (NC * NS)
    npad = _npad(n_nodes)
    mesh = plsc.VectorSubcoreMesh(core_axis_name="c", subcore_axis_name="s")
    body = functools.partial(_sc_msg_body, n_nodes, pack, cpw, True)
    f = pl.kernel(
        body,
        out_type=[jax.ShapeDtypeStruct((NC, npad, WIDTH), jnp.float32),
                  jax.ShapeDtypeStruct((n_nodes,), jnp.float32),
                  jax.ShapeDtypeStruct((n_nodes,), jnp.float32)],
        mesh=mesh,
        scratch_types=[
            pltpu.VMEM((2, CHUNK), jnp.int32),
            pltpu.VMEM((pack, CHUNK), jnp.float32),
            pltpu.VMEM((CHUNK, WIDTH), jnp.float32),
            pltpu.VMEM((CHUNK,), jnp.float32),
            pltpu.VMEM_SHARED((npad, WIDTH), jnp.float32),
            pltpu.VMEM_SHARED((n_nodes,), jnp.float32),
            pltpu.SemaphoreType.DMA,
        ],
    )
    return f(yflat, meta, bas, zeros2d, zeros1)


def kernel(adj, x, pseudo, Ws, roots, biases, lin1_w, lin1_b, lin2_w, lin2_b):
    src = adj[0]
    dst = adj[1]
    E = src.shape[0]
    N = x.shape[0]

    basis, kidx = _spline_basis(pseudo)

    P = E * 8
    kidxf = kidx.reshape(-1)
    srcf = jnp.broadcast_to(src[:, None], (E, 8)).reshape(-1)
    dstf = jnp.broadcast_to(dst[:, None], (E, 8)).reshape(-1)
    basf = basis.reshape(-1)
    cgrain = NC * NS * CHUNK
    cpw = (P + cgrain - 1) // cgrain
    Ppad = cpw * cgrain
    pad = Ppad - P
    if pad:
        extra = jnp.arange(pad, dtype=jnp.int32)
        kidxf = jnp.concatenate([kidxf, extra % K3])
        srcf = jnp.concatenate([srcf, extra % N])
        dstf = jnp.concatenate([dstf, extra % N])
        basf = jnp.concatenate([basf, jnp.zeros((pad,), jnp.float32)])
    NCH = Ppad // CHUNK

    def make_meta(ntp, pack):
        rowq = kidxf * ntp + srcf // pack
        meta = jnp.stack([rowq, dstf])
        meta = meta.reshape(2, NCH, CHUNK).transpose(1, 0, 2)
        sub = srcf % pack
        streams = [jnp.where(sub == j, basf, 0.0) for j in range(pack)]
        bas = jnp.stack(streams)
        bas = bas.reshape(pack, NCH, CHUNK).transpose(1, 0, 2)
        return meta, bas

    NT0 = 10016
    NTP0 = NT0 // 4
    NH = N // 2

    meta4, bas4 = make_meta(NTP0, 4)
    meta2, bas2 = make_meta(NH, 2)

    npad = _npad(N)
    zeros128 = jnp.zeros((npad, WIDTH), jnp.float32)
    zeros1 = jnp.zeros((N,), jnp.float32)

    xs = [jnp.pad(x[j::4], ((0, NTP0 - (N + 3 - j) // 4), (0, 0)))
          for j in range(4)]
    y0 = _expand0(xs, Ws[0])
    msg, d0, d1 = _sc_msg_deg(y0, meta4, bas4, zeros128, zeros1, N, 4)
    deg = jnp.stack([d0, d1])[:, :, None]

    hprev = x
    pack_prev, dout_prev = 4, 32
    for li in range(1, len(Ws)):
        din, dout = Ws[li].shape[1], Ws[li].shape[2]
        h, he, ho = _combine(msg, deg, hprev, roots[li - 1], biases[li - 1],
                             pack_prev, dout_prev)
        y = _expand2(he, ho, Ws[li])
        msg = _sc_msg(y, meta2, bas2, zeros128, N, 2)
        hprev = h
        pack_prev, dout_prev = 2, dout

    return _final(msg, deg, hprev, roots[-1], biases[-1],
                  lin1_w, lin1_b, lin2_w, lin2_b, pack_prev, dout_prev)


# speedup vs baseline: 4.9297x; 1.5927x over previous
import functools

import jax
import jax.numpy as jnp
import numpy as np
from jax import lax
from jax.experimental import pallas as pl
from jax.experimental.pallas import tpu as pltpu
from jax.experimental.pallas import tpu_sc as plsc

KS = 5
DIM = 3
K3 = KS ** DIM

NC = 2
NS = 16
LANES = 16
CHUNK = 128
WIDTH = 128


def _basis_kernel(p_ref, basis_ref, kidx_ref):
    v = p_ref[...] * (KS - 1)
    bot = jnp.floor(v)
    frac = v - bot
    boti = bot.astype(jnp.int32)
    bcols = []
    kcols = []
    for s in range(8):
        w = None
        ki = None
        for d in range(DIM):
            bit = (s >> d) & 1
            f = frac[:, d:d + 1]
            wd = f if bit else 1.0 - f
            w = wd if w is None else w * wd
            idx = jnp.clip(boti[:, d:d + 1] + bit, 0, KS - 1)
            contrib = idx * (KS ** d)
            ki = contrib if ki is None else ki + contrib
        bcols.append(w)
        kcols.append(ki)
    basis_ref[...] = jnp.concatenate(bcols, axis=1)
    kidx_ref[...] = jnp.concatenate(kcols, axis=1)


def _spline_basis(pseudo):
    E = pseudo.shape[0]
    B = 2000
    return pl.pallas_call(
        _basis_kernel,
        grid=(E // B,),
        in_specs=[pl.BlockSpec((B, DIM), lambda i: (i, 0))],
        out_specs=[pl.BlockSpec((B, 8), lambda i: (i, 0)),
                   pl.BlockSpec((B, 8), lambda i: (i, 0))],
        out_shape=[jax.ShapeDtypeStruct((E, 8), jnp.float32),
                   jax.ShapeDtypeStruct((E, 8), jnp.int32)],
    )(pseudo)


def _elu(x):
    return jnp.where(x > 0, x, jnp.exp(x) - 1.0)


def _expand0_kernel(x0_ref, x1_ref, x2_ref, x3_ref, w_ref, y_ref):
    w = w_ref[0]
    y_ref[...] = jnp.concatenate(
        [x0_ref[...] * w, x1_ref[...] * w, x2_ref[...] * w, x3_ref[...] * w],
        axis=1)


def _expand0(xs, W):
    NTP = xs[0].shape[0]
    return pl.pallas_call(
        _expand0_kernel,
        grid=(K3,),
        in_specs=[pl.BlockSpec((NTP, 1), lambda k: (0, 0))] * 4
        + [pl.BlockSpec((1, 1, 32), lambda k: (k, 0, 0))],
        out_specs=pl.BlockSpec((NTP, WIDTH), lambda k: (k, 0)),
        out_shape=jax.ShapeDtypeStruct((K3 * NTP, WIDTH), jnp.float32),
    )(*xs, W)


def _expand2_kernel(he_ref, ho_ref, w_ref, y_ref):
    w = w_ref[0]
    ye = jnp.dot(he_ref[...], w, preferred_element_type=jnp.float32)
    yo = jnp.dot(ho_ref[...], w, preferred_element_type=jnp.float32)
    y_ref[...] = jnp.concatenate([ye, yo], axis=1)


def _expand2(he, ho, W):
    NH, din = he.shape
    return pl.pallas_call(
        _expand2_kernel,
        grid=(K3,),
        in_specs=[pl.BlockSpec((NH, din), lambda k: (0, 0)),
                  pl.BlockSpec((NH, din), lambda k: (0, 0)),
                  pl.BlockSpec((1, din, 64), lambda k: (k, 0, 0))],
        out_specs=pl.BlockSpec((NH, WIDTH), lambda k: (k, 0)),
        out_shape=jax.ShapeDtypeStruct((K3 * NH, WIDTH), jnp.float32),
    )(he, ho, W)


def _fold(msg_blk, pack, dout):
    msum = msg_blk[0] + msg_blk[1]
    acc = msum[:, 0:dout]
    for j in range(1, pack):
        acc = acc + msum[:, j * dout:(j + 1) * dout]
    return acc


def _combine_kernel(pack, dout, msg_ref, deg_ref, hprev_ref, root_ref,
                    bias_ref, h_ref, he_ref, ho_ref):
    m = _fold(msg_ref[...], pack, dout)
    dsum = deg_ref[0] + deg_ref[1]
    invd = 1.0 / jnp.maximum(dsum, 1.0)
    h = m * invd + jnp.dot(hprev_ref[...], root_ref[...],
                           preferred_element_type=jnp.float32)
    h = _elu(h + bias_ref[...])
    h_ref[...] = h
    nb = h.shape[0]
    h3 = h.reshape(nb // 2, 2, dout)
    he_ref[...] = h3[:, 0, :]
    ho_ref[...] = h3[:, 1, :]


def _combine(msg, deg, hprev, root, bias, pack, dout):
    N = hprev.shape[0]
    dpp = hprev.shape[1]
    NB = 400
    kfn = functools.partial(_combine_kernel, pack, dout)
    return pl.pallas_call(
        kfn,
        grid=(N // NB,),
        in_specs=[
            pl.BlockSpec((2, NB, WIDTH), lambda i: (0, i, 0)),
            pl.BlockSpec((2, NB, 1), lambda i: (0, i, 0)),
            pl.BlockSpec((NB, dpp), lambda i: (i, 0)),
            pl.BlockSpec((dpp, dout), lambda i: (0, 0)),
            pl.BlockSpec((1, dout), lambda i: (0, 0)),
        ],
        out_specs=[pl.BlockSpec((NB, dout), lambda i: (i, 0)),
                   pl.BlockSpec((NB // 2, dout), lambda i: (i, 0)),
                   pl.BlockSpec((NB // 2, dout), lambda i: (i, 0))],
        out_shape=[jax.ShapeDtypeStruct((N, dout), jnp.float32),
                   jax.ShapeDtypeStruct((N // 2, dout), jnp.float32),
                   jax.ShapeDtypeStruct((N // 2, dout), jnp.float32)],
    )(msg, deg, hprev, root, bias.reshape(1, -1))


def _final_kernel(pack, dout, msg_ref, deg_ref, hprev_ref, root_ref, bias_ref,
                  w1_ref, b1_ref, w2_ref, b2_ref, o_ref):
    m = _fold(msg_ref[...], pack, dout)
    dsum = deg_ref[0] + deg_ref[1]
    invd = 1.0 / jnp.maximum(dsum, 1.0)
    h = m * invd + jnp.dot(hprev_ref[...], root_ref[...],
                           preferred_element_type=jnp.float32)
    h = _elu(h + bias_ref[...])
    h = _elu(jnp.dot(h, w1_ref[...], preferred_element_type=jnp.float32)
             + b1_ref[...])
    z = jnp.dot(h, w2_ref[...], preferred_element_type=jnp.float32) + b2_ref[...]
    mx = jnp.max(z, axis=-1, keepdims=True)
    zs = z - mx
    lse = jnp.log(jnp.sum(jnp.exp(zs), axis=-1, keepdims=True))
    o_ref[...] = zs - lse


def _final(msg, deg, hprev, root, bias, w1, b1, w2, b2, pack, dout):
    N = hprev.shape[0]
    dpp = hprev.shape[1]
    NB = 400
    kfn = functools.partial(_final_kernel, pack, dout)
    return pl.pallas_call(
        kfn,
        grid=(N // NB,),
        in_specs=[
            pl.BlockSpec((2, NB, WIDTH), lambda i: (0, i, 0)),
            pl.BlockSpec((2, NB, 1), lambda i: (0, i, 0)),
            pl.BlockSpec((NB, dpp), lambda i: (i, 0)),
            pl.BlockSpec((dpp, dout), lambda i: (0, 0)),
            pl.BlockSpec((1, dout), lambda i: (0, 0)),
            pl.BlockSpec(w1.shape, lambda i: (0, 0)),
            pl.BlockSpec((1, w1.shape[1]), lambda i: (0, 0)),
            pl.BlockSpec(w2.shape, lambda i: (0, 0)),
            pl.BlockSpec((1, w2.shape[1]), lambda i: (0, 0)),
        ],
        out_specs=pl.BlockSpec((NB, w2.shape[1]), lambda i: (i, 0)),
        out_shape=jax.ShapeDtypeStruct((N, w2.shape[1]), jnp.float32),
    )(msg, deg, hprev, root, bias.reshape(1, -1), w1, b1.reshape(1, -1),
      w2, b2.reshape(1, -1))


def _bcast_lane(vec, i):
    return lax.broadcast_in_dim(lax.slice(vec, (i,), (i + 1,)), (LANES,), (0,))


def _npad(n_nodes):
    g = NS * 8
    return ((n_nodes + g - 1) // g) * g


def _sc_msg_body(n_nodes, pack, cpw, with_deg, *refs):
    if with_deg:
        (y_hbm, meta_hbm, bas_hbm, zeros_hbm, zeros1_hbm, out_hbm,
         deg0_hbm, deg1_hbm,
         metav, basv, rows, basisf, acc, dacc, sem) = refs
    else:
        (y_hbm, meta_hbm, bas_hbm, zeros_hbm, out_hbm,
         metav, basv, rows, acc, sem) = refs
    c = lax.axis_index("c")
    s = lax.axis_index("s")
    wid = s * NC + c
    rpt = _npad(n_nodes) // NS
    dout = WIDTH // pack

    pltpu.sync_copy(zeros_hbm.at[pl.ds(s * rpt, rpt)],
                    acc.at[pl.ds(s * rpt, rpt)])
    if with_deg:
        @pl.when(s == 0)
        def _zero_deg():
            pltpu.sync_copy(zeros1_hbm, dacc)
    plsc.subcore_barrier()

    def chunk_body(i, carry):
        base = wid * cpw + i
        pltpu.sync_copy(meta_hbm.at[base], metav)
        pltpu.sync_copy(bas_hbm.at[base], basv)
        pltpu.async_copy(y_hbm.at[metav.at[0]], rows, sem).wait()
        for g in range(CHUNK // LANES):
            bvs = [basv[j, pl.ds(g * LANES, LANES)] for j in range(pack)]
            if with_deg:
                bsum = bvs[0]
                for j in range(1, pack):
                    bsum = bsum + bvs[j]
                basisf[pl.ds(g * LANES, LANES)] = bsum
            for i2 in range(LANES):
                r = g * LANES + i2
                for j in range(pack):
                    b = _bcast_lane(bvs[j], i2)
                    for t in range(dout // LANES):
                        col = j * dout + t * LANES
                        rows[r, pl.ds(col, LANES)] = (
                            rows[r, pl.ds(col, LANES)] * b)
        pltpu.sync_copy(rows, acc.at[metav.at[1]], add=True)
        if with_deg:
            pltpu.sync_copy(basisf, dacc.at[metav.at[1]], add=True)
        return carry

    lax.fori_loop(0, cpw, chunk_body, 0)
    plsc.subcore_barrier()

    pltpu.sync_copy(acc.at[pl.ds(s * rpt, rpt)],
                    out_hbm.at[c, pl.ds(s * rpt, rpt)])
    if with_deg:
        @pl.when((s == 0) & (c == 0))
        def _copy_deg0():
            pltpu.sync_copy(dacc, deg0_hbm)

        @pl.when((s == 0) & (c == 1))
        def _copy_deg1():
            pltpu.sync_copy(dacc, deg1_hbm)


def _scale_chunk(rowsr, basr, b, m, pack):
    dout = WIDTH // pack

    def gbody(g, carry):
        bvs = [basr[m, j, pl.ds(g * LANES, LANES)] for j in range(pack)]
        for i2 in range(LANES):
            r = g * LANES + i2
            for j in range(pack):
                bsp = _bcast_lane(bvs[j], i2)
                for t in range(dout // LANES):
                    col = j * dout + t * LANES
                    rowsr[b, r, pl.ds(col, LANES)] = (
                        rowsr[b, r, pl.ds(col, LANES)] * bsp)
        return carry

    lax.fori_loop(0, CHUNK // LANES, gbody, 0)


def _sc_msg_pipe_body(n_nodes, pack, cpw, nch, *refs):
    (y_hbm, meta_hbm, bas_hbm, zeros_hbm, out_hbm,
     metam, basm, rows, acc, *sems) = refs
    semm = sems[0:8]
    semg = sems[8:10]
    semsc = sems[10:12]
    c = lax.axis_index("c")
    s = lax.axis_index("s")
    wid = s * NC + c
    rpt = _npad(n_nodes) // NS
    base0 = wid * cpw

    pltpu.sync_copy(zeros_hbm.at[pl.ds(s * rpt, rpt)],
                    acc.at[pl.ds(s * rpt, rpt)])
    plsc.subcore_barrier()

    def meta_fetch(chunk, m, sem):
        pltpu.async_copy(meta_hbm.at[chunk], metam.at[m], sem)
        pltpu.async_copy(bas_hbm.at[chunk], basm.at[m], sem)

    def meta_wait(m, sem):
        pltpu.make_async_copy(meta_hbm.at[0], metam.at[m], sem).wait()
        pltpu.make_async_copy(bas_hbm.at[0], basm.at[m], sem).wait()

    def gather(chunk_meta_m, b):
        pltpu.async_copy(y_hbm.at[metam.at[chunk_meta_m, 0]], rows.at[b],
                         semg[b])

    def gather_wait(b):
        pltpu.make_async_copy(y_hbm.at[metam.at[0, 0]], rows.at[b],
                              semg[b]).wait()

    def scatter(m, b):
        pltpu.async_copy(rows.at[b], acc.at[metam.at[m, 1]], semsc[b],
                         add=True)

    def scatter_wait(b):
        pltpu.make_async_copy(rows.at[b], acc.at[metam.at[0, 1]],
                              semsc[b]).wait()

    for u in range(4):
        meta_fetch(base0 + u, u, semm[u])
    pltpu.async_copy(meta_hbm.at[base0], metam.at[7], semm[7])
    pltpu.async_copy(zeros_hbm.at[pl.ds(0, pack)], basm.at[7], semm[7])
    pltpu.async_copy(zeros_hbm.at[pl.ds(0, CHUNK)], rows.at[1], semg[1])
    pltpu.async_copy(zeros_hbm.at[pl.ds(0, CHUNK)], rows.at[0], semsc[0])
    meta_wait(7, semm[7])

    def group(g, carry):
        for u in range(8):
            cl = 8 * g + u
            b = u % 2
            m = u
            bp = (u - 1) % 2
            mp_ = (u - 1) % 8
            meta_wait(m, semm[m])
            scatter_wait(b)
            cf = jnp.minimum(base0 + cl + 4, nch - 1)
            meta_fetch(cf, (u + 4) % 8, semm[(u + 4) % 8])
            gather(m, b)
            gather_wait(bp)
            _scale_chunk(rows, basm, bp, mp_, pack)
            scatter(mp_, bp)
        return carry

    lax.fori_loop(0, cpw // 8, group, 0)

    gather_wait(1)
    _scale_chunk(rows, basm, 1, 7, pack)
    scatter(7, 1)
    for b in range(2):
        scatter_wait(b)
    for u in range(4):
        meta_wait(u, semm[u])
    plsc.subcore_barrier()

    pltpu.sync_copy(acc.at[pl.ds(s * rpt, rpt)],
                    out_hbm.at[c, pl.ds(s * rpt, rpt)])


@functools.partial(jax.jit, static_argnums=(4, 5))
def _sc_msg(yflat, meta, bas, zeros2d, n_nodes, pack):
    nchunks = meta.shape[0]
    cpw = nchunks // (NC * NS)
    npad = _npad(n_nodes)
    mesh = plsc.VectorSubcoreMesh(core_axis_name="c", subcore_axis_name="s")
    body = functools.partial(_sc_msg_pipe_body, n_nodes, pack, cpw, nchunks)
    f = pl.kernel(
        body,
        out_type=jax.ShapeDtypeStruct((NC, npad, WIDTH), jnp.float32),
        mesh=mesh,
        scratch_types=[
            pltpu.VMEM((8, 2, CHUNK), jnp.int32),
            pltpu.VMEM((8, pack, CHUNK), jnp.float32),
            pltpu.VMEM((2, CHUNK, WIDTH), jnp.float32),
            pltpu.VMEM_SHARED((npad, WIDTH), jnp.float32),
        ] + [pltpu.SemaphoreType.DMA] * 12,
    )
    return f(yflat, meta, bas, zeros2d)


@functools.partial(jax.jit, static_argnums=(5, 6))
def _sc_msg_deg(yflat, meta, bas, zeros2d, zeros1, n_nodes, pack):
    nchunks = meta.shape[0]
    cpw = nchunks // (NC * NS)
    npad = _npad(n_nodes)
    mesh = plsc.VectorSubcoreMesh(core_axis_name="c", subcore_axis_name="s")
    body = functools.partial(_sc_msg_body, n_nodes, pack, cpw, True)
    f = pl.kernel(
        body,
        out_type=[jax.ShapeDtypeStruct((NC, npad, WIDTH), jnp.float32),
                  jax.ShapeDtypeStruct((n_nodes,), jnp.float32),
                  jax.ShapeDtypeStruct((n_nodes,), jnp.float32)],
        mesh=mesh,
        scratch_types=[
            pltpu.VMEM((2, CHUNK), jnp.int32),
            pltpu.VMEM((pack, CHUNK), jnp.float32),
            pltpu.VMEM((CHUNK, WIDTH), jnp.float32),
            pltpu.VMEM((CHUNK,), jnp.float32),
            pltpu.VMEM_SHARED((npad, WIDTH), jnp.float32),
            pltpu.VMEM_SHARED((n_nodes,), jnp.float32),
            pltpu.SemaphoreType.DMA,
        ],
    )
    return f(yflat, meta, bas, zeros2d, zeros1)


def kernel(adj, x, pseudo, Ws, roots, biases, lin1_w, lin1_b, lin2_w, lin2_b):
    src = adj[0]
    dst = adj[1]
    E = src.shape[0]
    N = x.shape[0]

    basis, kidx = _spline_basis(pseudo)

    P = E * 8
    kidxf = kidx.reshape(-1)
    srcf = jnp.broadcast_to(src[:, None], (E, 8)).reshape(-1)
    dstf = jnp.broadcast_to(dst[:, None], (E, 8)).reshape(-1)
    basf = basis.reshape(-1)
    cgrain = NC * NS * CHUNK
    cpw = (P + cgrain - 1) // cgrain
    cpw = ((cpw + 7) // 8) * 8
    Ppad = cpw * cgrain
    pad = Ppad - P
    if pad:
        extra = jnp.arange(pad, dtype=jnp.int32)
        kidxf = jnp.concatenate([kidxf, extra % K3])
        srcf = jnp.concatenate([srcf, extra % N])
        dstf = jnp.concatenate([dstf, extra % N])
        basf = jnp.concatenate([basf, jnp.zeros((pad,), jnp.float32)])
    NCH = Ppad // CHUNK

    def make_meta(ntp, pack):
        rowq = kidxf * ntp + srcf // pack
        meta = jnp.stack([rowq, dstf])
        meta = meta.reshape(2, NCH, CHUNK).transpose(1, 0, 2)
        sub = srcf % pack
        streams = [jnp.where(sub == j, basf, 0.0) for j in range(pack)]
        bas = jnp.stack(streams)
        bas = bas.reshape(pack, NCH, CHUNK).transpose(1, 0, 2)
        return meta, bas

    NT0 = 10016
    NTP0 = NT0 // 4
    NH = N // 2

    meta4, bas4 = make_meta(NTP0, 4)
    meta2, bas2 = make_meta(NH, 2)

    npad = _npad(N)
    zeros128 = jnp.zeros((npad, WIDTH), jnp.float32)
    zeros1 = jnp.zeros((N,), jnp.float32)

    xs = [jnp.pad(x[j::4], ((0, NTP0 - (N + 3 - j) // 4), (0, 0)))
          for j in range(4)]
    y0 = _expand0(xs, Ws[0])
    msg, d0, d1 = _sc_msg_deg(y0, meta4, bas4, zeros128, zeros1, N, 4)
    deg = jnp.stack([d0, d1])[:, :, None]

    hprev = x
    pack_prev, dout_prev = 4, 32
    for li in range(1, len(Ws)):
        din, dout = Ws[li].shape[1], Ws[li].shape[2]
        h, he, ho = _combine(msg, deg, hprev, roots[li - 1], biases[li - 1],
                             pack_prev, dout_prev)
        y = _expand2(he, ho, Ws[li])
        msg = _sc_msg(y, meta2, bas2, zeros128, N, 2)
        hprev = h
        pack_prev, dout_prev = 2, dout

    return _final(msg, deg, hprev, roots[-1], biases[-1],
                  lin1_w, lin1_b, lin2_w, lin2_b, pack_prev, dout_prev)
